# baseline (speedup 1.0000x reference)
import functools

import jax
import jax.numpy as jnp
from jax import lax
from jax.experimental import pallas as pl
from jax.experimental.pallas import tpu as pltpu
from jax.experimental.pallas import tpu_sc as plsc

N_ROWS = 262144
N_COLS = 128
NUM_CORES = 2
NUM_SUBCORES = 16
NUM_WORKERS = NUM_CORES * NUM_SUBCORES
RPW = N_ROWS // NUM_WORKERS
L = 16
E_ROWS = 2048
R_ROWS = RPW - E_ROWS
NCHUNK = 6
CH = R_ROWS // NCHUNK
NSET = 3


def _dg(v, idx):
    return lax.gather(
        v,
        idx.reshape(L, 1),
        lax.GatherDimensionNumbers(
            offset_dims=(), collapsed_slice_dims=(0,), start_index_map=(0,)
        ),
        slice_sizes=(1,),
        mode=lax.GatherScatterMode.PROMISE_IN_BOUNDS,
    )


def _action(dx_s, dy_s, clk, dist_x, dist_y):
    go_down = dy_s > 1
    go_right = dx_s > 0
    punch = (clk % 12) < 4
    d2 = dist_y <= 2
    act = jnp.where(go_down, 5, 2)
    act = jnp.where(d2 & (dist_x > 26), jnp.where(go_right, 3, 4), act)
    act = jnp.where(d2 & (dist_x < 23), jnp.where(go_right, 4, 3), act)
    act = jnp.where(
        d2 & (dist_x >= 23) & (dist_x <= 26), jnp.where(punch, 1, 0), act
    )
    return act


def _sc_body(view8_hbm, viewf_hbm, out_hbm, *refs):
    (ixy0, ixy1, ixy2, iclk0, iclk1, iclk2,
     bxy0, bxy1, bxy2, bclk0, bclk1, bclk2,
     imx, isx, imy, isy, ick, bmx, bsx, bmy, bsy, bck,
     acc_v, semr, seme) = refs
    idx_sets = ((ixy0, iclk0), (ixy1, iclk1), (ixy2, iclk2))
    buf_sets = ((bxy0, bclk0), (bxy1, bclk1), (bxy2, bclk2))
    eidx = (imx, isx, imy, isy, ick)
    ebuf = (bmx, bsx, bmy, bsy, bck)

    cid = lax.axis_index("c")
    sid = lax.axis_index("s")
    wid = sid * NUM_CORES + cid
    row0 = wid * RPW
    rrow0 = row0 + E_ROWS
    iot = lax.iota(jnp.int32, L)
    one = jnp.ones((L,), jnp.int32)
    swap = iot ^ 1
    lane0 = jnp.zeros((L,), jnp.int32)
    qmask = tuple((iot >> 2) == q for q in range(4))
    rots = tuple((iot - 4 * r) & 15 for r in (1, 2, 3))
    pat0 = (iot & 3) * 4
    pat2 = pat0 + 2

    def fill_row(c, idxs):
        ixy, iclk = idxs

        def body(j, carry):
            r8 = 8 * (rrow0 + c * CH + j * L + iot)
            ixy[pl.ds(j * L, L)] = r8 + 2
            iclk[pl.ds(j * L, L)] = r8
            return carry

        lax.fori_loop(0, CH // L, body, 0)

    def fire_row(idxs, bufs):
        ixy, iclk = idxs
        bxy, bclk = bufs
        pltpu.async_copy(view8_hbm.at[ixy], bxy, semr)
        pltpu.async_copy(view8_hbm.at[iclk], bclk, semr)

    def drain_row(idxs, bufs):
        ixy, iclk = idxs
        bxy, bclk = bufs
        pltpu.make_async_copy(view8_hbm.at[ixy], bxy, semr).wait()
        pltpu.make_async_copy(view8_hbm.at[iclk], bclk, semr).wait()

    def fill_elem():
        def body(j, carry):
            base = N_COLS * (row0 + j * L + iot)
            imx[pl.ds(j * L, L)] = base + 32
            isx[pl.ds(j * L, L)] = base + 33
            imy[pl.ds(j * L, L)] = base + 34
            isy[pl.ds(j * L, L)] = base + 35
            ick[pl.ds(j * L, L)] = base
            return carry

        lax.fori_loop(0, E_ROWS // L, body, 0)

    def fire_elem():
        for idx, buf in zip(eidx, ebuf):
            pltpu.async_copy(viewf_hbm.at[idx], buf, seme)

    def drain_elem():
        for idx, buf in zip(eidx, ebuf):
            pltpu.make_async_copy(viewf_hbm.at[idx], buf, seme).wait()

    def compute_row(bufs, acc):
        bxy, bclk = bufs

        def group(g, acc2):
            dx = lane0
            dy = lane0
            ck = lane0
            for q in range(4):
                b = g * L + 4 * q
                m = bxy[b, :]
                mc = bclk[b, :]
                for r in (1, 2, 3):
                    m = jnp.where(qmask[r], _dg(bxy[b + r, :], rots[r - 1]), m)
                    mc = jnp.where(qmask[r], _dg(bclk[b + r, :], rots[r - 1]), mc)
                d = _dg(m, swap) - m
                dx = jnp.where(qmask[q], _dg(d, pat0), dx)
                dy = jnp.where(qmask[q], _dg(d, pat2), dy)
                ck = jnp.where(qmask[q], _dg(mc, pat0), ck)
            act = _action(dx, dy, ck, jnp.abs(dx), jnp.abs(dy))
            return acc2 | (one << act)

        return lax.fori_loop(0, CH // L, group, acc)

    def compute_elem(acc):
        def body16(i, acc2):
            s = pl.ds(i * L, L)
            mi_x = bmx[s].astype(jnp.int32)
            su_x = bsx[s].astype(jnp.int32)
            mi_y = bmy[s].astype(jnp.int32)
            su_y = bsy[s].astype(jnp.int32)
            clk = bck[s].astype(jnp.int32)
            dxs = su_x - mi_x
            dys = su_y - mi_y
            act = _action(dxs, dys, clk, jnp.abs(dxs), jnp.abs(dys))
            return acc2 | (one << act)

        return lax.fori_loop(0, E_ROWS // L, body16, acc)

    for c in range(NSET):
        fill_row(c, idx_sets[c])
        fire_row(idx_sets[c], buf_sets[c])
    acc = jnp.zeros((L,), jnp.int32)
    for c in range(NCHUNK):
        s = c % NSET
        drain_row(idx_sets[s], buf_sets[s])
        acc = compute_row(buf_sets[s], acc)
        if c + NSET < NCHUNK:
            fill_row(c + NSET, idx_sets[s])
            fire_row(idx_sets[s], buf_sets[s])
        if c == NCHUNK - NSET:
            fill_elem()
            fire_elem()
    drain_elem()
    acc = compute_elem(acc)

    acc_v[...] = acc
    pltpu.sync_copy(acc_v, out_hbm.at[wid])


@jax.jit
def _run(ram):
    view8 = ram.reshape(N_ROWS * 8, L)
    viewf = lax.bitcast_convert_type(ram, jnp.uint32).reshape(N_ROWS * N_COLS)
    mesh = plsc.VectorSubcoreMesh(core_axis_name="c", subcore_axis_name="s")
    scratch = (
        [pltpu.VMEM((CH,), jnp.int32) for _ in range(6)]
        + [pltpu.VMEM((CH, L), jnp.int32) for _ in range(6)]
        + [pltpu.VMEM((E_ROWS,), jnp.int32) for _ in range(5)]
        + [pltpu.VMEM((E_ROWS,), jnp.uint32) for _ in range(5)]
        + [pltpu.VMEM((L,), jnp.int32)]
        + [pltpu.SemaphoreType.DMA, pltpu.SemaphoreType.DMA]
    )
    k = functools.partial(
        pl.kernel,
        mesh=mesh,
        out_type=jax.ShapeDtypeStruct((NUM_WORKERS, L), jnp.int32),
        scratch_types=scratch,
        compiler_params=pltpu.CompilerParams(use_tc_tiling_on_sc=False),
    )(_sc_body)
    masks = k(view8, viewf)
    bits = (masks[:, :, None] >> jnp.arange(6, dtype=jnp.int32)) & 1
    seen = jnp.max(bits, axis=(0, 1)).astype(jnp.float32)
    logits = jnp.zeros((1, 18), dtype=jnp.float32)
    return lax.dynamic_update_slice(logits, seen.reshape(1, 6), (0, 0))


def kernel(ram):
    return _run(ram)
